# baseline (speedup 1.0000x reference)
import functools

import jax
import jax.numpy as jnp
from jax.experimental import pallas as pl
from jax.experimental.pallas import tpu as pltpu

_TI = 32
_G = 2


def _ec_weights(C, W1, b1, W2, b2):
    H = W2.shape[1]
    TI, G = _TI, _G
    WabT = (W1[:C] - W1[C:2 * C]).T
    WbT = W1[C:2 * C].T
    WcT = W1[2 * C:].T
    eye = jnp.eye(TI, dtype=jnp.float32)
    W1blk = jnp.concatenate(
        [jnp.kron(eye, WcT), jnp.tile(WbT, (TI, 1))], axis=1)
    W2blk = jnp.kron(jnp.eye(G, dtype=jnp.float32), W2.T)
    b2G = jnp.tile(b2.reshape(H, 1), (G, 1))
    return WabT, W1blk, b1.reshape(H, 1), W2blk, b2G


def _edge_conv_tile(TI, G, H, NG, getx, xti, adj, eTs,
                    WabT, W1blk, b1T, W2blk, b2G):
    xT = jnp.concatenate([getx(g) for g in range(NG)], axis=1)
    AT = WabT @ xti + b1T
    big = W1blk @ jnp.concatenate([eTs, xT], axis=0)
    Acol = jnp.concatenate([AT[:, t:t + 1] for t in range(TI)], axis=0)
    h1 = jnp.maximum(big + Acol, 0.0)
    pen = jnp.where(adj > 0.0, 0.0, jnp.float32(-1e30))
    cols = []
    for g in range(TI // G):
        h2 = jnp.maximum(
            W2blk @ h1[g * G * H:(g + 1) * G * H, :] + b2G, 0.0)
        for u in range(G):
            t = g * G + u
            hm = h2[u * H:(u + 1) * H, :] + pen[t:t + 1, :]
            agg = jnp.max(hm, axis=1, keepdims=True)
            cols.append(jnp.where(agg <= -1e29, 0.0, agg))
    return jnp.concatenate(cols, axis=1)


def _mega_body(TI, G, H, H2,
               xq_ref, adj_ref, eT_ref,
               Wab1_ref, W1blk1_ref, b11_ref, W2blk1_ref, b21_ref,
               Wab2_ref, W1blk2_ref, b12_ref, W2blk2_ref, b22_ref,
               oht_ref, ohh_ref, WaT_ref, WbT_ref, b3T_ref, W4sel_ref,
               b4_ref, out_ref, y_scr, z_scr):
    s = pl.program_id(0)
    b = pl.program_id(1)
    i = pl.program_id(2)
    TC2 = eT_ref.shape[1] * eT_ref.shape[2]
    V = eT_ref.shape[3]
    NG = xq_ref.shape[1]

    @pl.when(s == 0)
    def _stage0():
        eTs = eT_ref[0].reshape(TC2, V)
        y_scr[b, i] = _edge_conv_tile(
            TI, G, H, NG, lambda g: xq_ref[0, g], xq_ref[0, i],
            adj_ref[0], eTs,
            Wab1_ref[...], W1blk1_ref[...], b11_ref[...],
            W2blk1_ref[...], b21_ref[...])

    @pl.when(s == 1)
    def _stage1():
        eTs = eT_ref[0].reshape(TC2, V)
        z_scr[b, i] = _edge_conv_tile(
            TI, G, H, NG, lambda g: y_scr[b, g], y_scr[b, i],
            adj_ref[0], eTs,
            Wab2_ref[...], W1blk2_ref[...], b12_ref[...],
            W2blk2_ref[...], b22_ref[...])

    @pl.when(s == 2)
    def _stage2():
        PT = jnp.concatenate(
            [WaT_ref[...] @ z_scr[b, g] for g in range(NG)], axis=1)
        QT = WbT_ref[...] @ z_scr[b, i] + b3T_ref[...]
        R1 = ohh_ref[...] @ QT
        Qcol = (R1 * oht_ref[...]) @ jnp.ones((TI, 1), jnp.float32)
        PTt = jnp.broadcast_to(PT[None], (TI, H2, V)).reshape(TI * H2, V)
        Hb = jnp.maximum(PTt + Qcol, 0.0)
        ot = W4sel_ref[...] @ Hb + b4_ref[...]
        out_ref[0] = jax.nn.sigmoid(ot)


def kernel(adjacency, node_features, edge_attributes,
           ec1_W1, ec1_b1, ec1_W2, ec1_b2,
           ec2_W1, ec2_b1, ec2_W2, ec2_b2,
           lin3_W, lin3_b, out_W, out_b):
    B, V, C = node_features.shape
    C2 = edge_attributes.shape[-1]
    H = ec1_W2.shape[1]
    H2 = lin3_W.shape[1]
    TI, G = _TI, _G
    NG = V // TI

    xq = node_features.reshape(B, NG, TI, C).transpose(0, 1, 3, 2)
    eT = edge_attributes.transpose(0, 1, 3, 2)

    w1 = _ec_weights(C, ec1_W1, ec1_b1, ec1_W2, ec1_b2)
    w2 = _ec_weights(H, ec2_W1, ec2_b1, ec2_W2, ec2_b2)
    WaT = lin3_W[:H].T
    WbT = lin3_W[H:].T
    r = jnp.arange(TI * H2)
    oht = (r[:, None] // H2 == jnp.arange(TI)[None, :]).astype(jnp.float32)
    ohh = (r[:, None] % H2 == jnp.arange(H2)[None, :]).astype(jnp.float32)
    W4sel = jnp.kron(jnp.eye(TI, dtype=jnp.float32), out_W.reshape(1, H2))

    def _w(shape):
        return pl.BlockSpec(shape, lambda s, b, i: (0,) * len(shape))

    TC2 = TI * C2
    return pl.pallas_call(
        functools.partial(_mega_body, TI, G, H, H2),
        grid=(3, B, NG),
        in_specs=[
            pl.BlockSpec((1, NG, C, TI), lambda s, b, i: (b, 0, 0, 0)),
            pl.BlockSpec((1, TI, V),
                         lambda s, b, i: (jnp.where(s == 2, 0, b),
                                          jnp.where(s == 2, 0, i), 0)),
            pl.BlockSpec((1, TI, C2, V),
                         lambda s, b, i: (jnp.where(s == 2, 0, b),
                                          jnp.where(s == 2, 0, i), 0, 0)),
            _w((H, C)), _w((TI * H, TC2 + C)), _w((H, 1)),
            _w((G * H, G * H)), _w((G * H, 1)),
            _w((H, H)), _w((TI * H, TC2 + H)), _w((H, 1)),
            _w((G * H, G * H)), _w((G * H, 1)),
            _w((TI * H2, TI)), _w((TI * H2, H2)),
            _w((H2, H)), _w((H2, H)), _w((H2, 1)),
            _w((TI, TI * H2)), _w((1, 1)),
        ],
        out_specs=pl.BlockSpec(
            (1, TI, V),
            lambda s, b, i: (jnp.where(s == 2, b, 0),
                             jnp.where(s == 2, i, 0), 0)),
        out_shape=jax.ShapeDtypeStruct((B, V, V), jnp.float32),
        scratch_shapes=[
            pltpu.VMEM((B, NG, H, TI), jnp.float32),
            pltpu.VMEM((B, NG, H, TI), jnp.float32),
        ],
        compiler_params=pltpu.CompilerParams(
            dimension_semantics=("arbitrary", "arbitrary", "arbitrary")),
    )(xq, adjacency, eT,
      *w1, *w2, oht, ohh, WaT, WbT, lin3_b.reshape(H2, 1), W4sel,
      out_b.reshape(1, 1))


# speedup vs baseline: 1.0306x; 1.0306x over previous
import functools

import jax
import jax.numpy as jnp
from jax.experimental import pallas as pl
from jax.experimental.pallas import tpu as pltpu

_TI = 32


_G = 2


def _ec_body(TI, G, H, xfq_ref, xq_ref, adj_ref, eT_ref,
             WabT_ref, WbT_ref, W1blk_ref, b1T_ref, W2blk_ref, b2G_ref,
             yq_ref):
    NG = xfq_ref.shape[1]
    V = NG * TI
    TC2 = eT_ref.shape[1] * eT_ref.shape[2]
    xfq = xfq_ref[0]
    xT = jnp.concatenate([xfq[g] for g in range(NG)], axis=1)
    AT = WabT_ref[...] @ xq_ref[0, 0] + b1T_ref[...]
    eTs = eT_ref[0].reshape(TC2, V)
    big = W1blk_ref[...] @ jnp.concatenate([eTs, xT], axis=0)
    Acol = jnp.concatenate([AT[:, t:t + 1] for t in range(TI)], axis=0)
    h1 = jnp.maximum(big + Acol, 0.0)
    pen = jnp.where(adj_ref[0] > 0.0, 0.0, jnp.float32(-1e30))
    cols = []
    for g in range(TI // G):
        h2 = jnp.maximum(
            W2blk_ref[...] @ h1[g * G * H:(g + 1) * G * H, :] + b2G_ref[...],
            0.0)
        for u in range(G):
            t = g * G + u
            hm = h2[u * H:(u + 1) * H, :] + pen[t:t + 1, :]
            agg = jnp.max(hm, axis=1, keepdims=True)
            cols.append(jnp.where(agg <= -1e29, 0.0, agg))
    yq_ref[0, 0] = jnp.concatenate(cols, axis=1)


def _edge_conv(adj, xq, eT, W1, b1, W2, b2):
    B, NG, C, TIx = xq.shape
    V = NG * TIx
    C2 = eT.shape[2]
    H = W2.shape[1]
    WabT = (W1[:C] - W1[C:2 * C]).T
    WbT = W1[C:2 * C].T
    WcT = W1[2 * C:].T
    TI, G = _TI, _G
    eye = jnp.eye(TI, dtype=jnp.float32)
    W1blk = jnp.concatenate(
        [jnp.kron(eye, WcT), jnp.tile(WbT, (TI, 1))], axis=1)
    W2blk = jnp.kron(jnp.eye(G, dtype=jnp.float32), W2.T)
    b2G = jnp.tile(b2.reshape(H, 1), (G, 1))
    return pl.pallas_call(
        functools.partial(_ec_body, TI, G, H),
        grid=(B, V // TI),
        in_specs=[
            pl.BlockSpec((1, NG, C, TI), lambda b, i: (b, 0, 0, 0)),
            pl.BlockSpec((1, 1, C, TI), lambda b, i: (b, i, 0, 0)),
            pl.BlockSpec((1, TI, V), lambda b, i: (b, i, 0)),
            pl.BlockSpec((1, TI, C2, V), lambda b, i: (b, i, 0, 0)),
            pl.BlockSpec((H, C), lambda b, i: (0, 0)),
            pl.BlockSpec((H, C), lambda b, i: (0, 0)),
            pl.BlockSpec((TI * H, TI * C2 + C), lambda b, i: (0, 0)),
            pl.BlockSpec((H, 1), lambda b, i: (0, 0)),
            pl.BlockSpec((G * H, G * H), lambda b, i: (0, 0)),
            pl.BlockSpec((G * H, 1), lambda b, i: (0, 0)),
        ],
        out_specs=pl.BlockSpec((1, 1, H, TI), lambda b, i: (b, i, 0, 0)),
        out_shape=jax.ShapeDtypeStruct((B, V // TI, H, TI), jnp.float32),
        compiler_params=pltpu.CompilerParams(
            dimension_semantics=("parallel", "parallel")),
    )(xq, xq, adj, eT, WabT, WbT, W1blk, b1.reshape(H, 1), W2blk, b2G)


def _fin_body(TI, zfq_ref, zq_ref, oht_ref, ohh_ref,
              WaT_ref, WbT_ref, b3T_ref, W4sel_ref, b4_ref, out_ref):
    NG = zfq_ref.shape[1]
    V = NG * TI
    H2 = WaT_ref.shape[0]
    zfq = zfq_ref[0]
    PT = jnp.concatenate(
        [WaT_ref[...] @ zfq[g] for g in range(NG)], axis=1)
    QT = WbT_ref[...] @ zq_ref[0, 0] + b3T_ref[...]
    Qcol = jnp.concatenate(
        [QT[:, t:t + 1] for t in range(TI)], axis=0)
    PTt = jnp.broadcast_to(PT[None], (TI, H2, V)).reshape(TI * H2, V)
    Hb = jnp.maximum(PTt + Qcol, 0.0)
    ot = W4sel_ref[...] @ Hb + b4_ref[...]
    out_ref[0] = jax.nn.sigmoid(ot)


def _edge_predict(zq, lin3_W, lin3_b, out_W, out_b):
    B, NG, C, TIx = zq.shape
    V = NG * TIx
    H2 = lin3_W.shape[1]
    TI = _TI
    WaT = lin3_W[:C].T
    WbT = lin3_W[C:].T
    r = jnp.arange(TI * H2)
    oht = (r[:, None] // H2 == jnp.arange(TI)[None, :]).astype(jnp.float32)
    ohh = (r[:, None] % H2 == jnp.arange(H2)[None, :]).astype(jnp.float32)
    W4sel = jnp.kron(jnp.eye(TI, dtype=jnp.float32), out_W.reshape(1, H2))
    return pl.pallas_call(
        functools.partial(_fin_body, TI),
        grid=(B, V // TI),
        in_specs=[
            pl.BlockSpec((1, NG, C, TI), lambda b, i: (b, 0, 0, 0)),
            pl.BlockSpec((1, 1, C, TI), lambda b, i: (b, i, 0, 0)),
            pl.BlockSpec((TI * H2, TI), lambda b, i: (0, 0)),
            pl.BlockSpec((TI * H2, H2), lambda b, i: (0, 0)),
            pl.BlockSpec((H2, C), lambda b, i: (0, 0)),
            pl.BlockSpec((H2, C), lambda b, i: (0, 0)),
            pl.BlockSpec((H2, 1), lambda b, i: (0, 0)),
            pl.BlockSpec((TI, TI * H2), lambda b, i: (0, 0)),
            pl.BlockSpec((1, 1), lambda b, i: (0, 0)),
        ],
        out_specs=pl.BlockSpec((1, TI, V), lambda b, i: (b, i, 0)),
        out_shape=jax.ShapeDtypeStruct((B, V, V), jnp.float32),
        compiler_params=pltpu.CompilerParams(
            dimension_semantics=("parallel", "parallel")),
    )(zq, zq, oht, ohh, WaT, WbT, lin3_b.reshape(H2, 1), W4sel,
      out_b.reshape(1, 1))


def kernel(adjacency, node_features, edge_attributes,
           ec1_W1, ec1_b1, ec1_W2, ec1_b2,
           ec2_W1, ec2_b1, ec2_W2, ec2_b2,
           lin3_W, lin3_b, out_W, out_b):
    B, V, C = node_features.shape
    TI = _TI
    xq = node_features.reshape(B, V // TI, TI, C).transpose(0, 1, 3, 2)
    eT = edge_attributes.transpose(0, 1, 3, 2)
    yq = _edge_conv(adjacency, xq, eT, ec1_W1, ec1_b1, ec1_W2, ec1_b2)
    zq = _edge_conv(adjacency, yq, eT, ec2_W1, ec2_b1, ec2_W2, ec2_b2)
    return _edge_predict(zq, lin3_W, lin3_b, out_W, out_b)


# speedup vs baseline: 1.1248x; 1.0914x over previous
import functools

import jax
import jax.numpy as jnp
from jax.experimental import pallas as pl
from jax.experimental.pallas import tpu as pltpu

_TI = 64
_TS = 32
_G = 2


def _ec_body(TI, TS, G, H, C2, xfq_ref, xq_ref, adj_ref, eT_ref,
             WabT_ref, W1blk_ref, b1T_ref, W2blk_ref, b2G_ref, yq_ref):
    NG = xfq_ref.shape[1]
    V = NG * TI
    xfq = xfq_ref[0]
    xT = jnp.concatenate([xfq[g] for g in range(NG)], axis=1)
    AT = WabT_ref[...] @ xq_ref[0, 0] + b1T_ref[...]
    eTs = eT_ref[0].reshape(TI * C2, V)
    pen = jnp.where(adj_ref[0] > 0.0, 0.0, jnp.float32(-1e30))
    cols = []
    for sub in range(TI // TS):
        eVs = eTs[sub * TS * C2:(sub + 1) * TS * C2]
        big = W1blk_ref[...] @ jnp.concatenate([eVs, xT], axis=0)
        Acol = jnp.concatenate(
            [AT[:, sub * TS + t:sub * TS + t + 1] for t in range(TS)],
            axis=0)
        h1 = jnp.maximum(big + Acol, 0.0)
        for g in range(TS // G):
            h2 = jnp.maximum(
                W2blk_ref[...] @ h1[g * G * H:(g + 1) * G * H, :]
                + b2G_ref[...], 0.0)
            for u in range(G):
                t = sub * TS + g * G + u
                hm = h2[u * H:(u + 1) * H, :] + pen[t:t + 1, :]
                agg = jnp.max(hm, axis=1, keepdims=True)
                cols.append(jnp.where(agg <= -1e29, 0.0, agg))
    yq_ref[0, 0] = jnp.concatenate(cols, axis=1)


def _edge_conv(adj, xq, eT, W1, b1, W2, b2):
    B, NG, C, TIx = xq.shape
    V = NG * TIx
    C2 = eT.shape[2]
    H = W2.shape[1]
    WabT = (W1[:C] - W1[C:2 * C]).T
    WbT = W1[C:2 * C].T
    WcT = W1[2 * C:].T
    TI, TS, G = _TI, _TS, _G
    eye = jnp.eye(TS, dtype=jnp.float32)
    W1blk = jnp.concatenate(
        [jnp.kron(eye, WcT), jnp.tile(WbT, (TS, 1))], axis=1)
    W2blk = jnp.kron(jnp.eye(G, dtype=jnp.float32), W2.T)
    b2G = jnp.tile(b2.reshape(H, 1), (G, 1))
    return pl.pallas_call(
        functools.partial(_ec_body, TI, TS, G, H, C2),
        grid=(B, V // TI),
        in_specs=[
            pl.BlockSpec((1, NG, C, TI), lambda b, i: (b, 0, 0, 0)),
            pl.BlockSpec((1, 1, C, TI), lambda b, i: (b, i, 0, 0)),
            pl.BlockSpec((1, TI, V), lambda b, i: (b, i, 0)),
            pl.BlockSpec((1, TI, C2, V), lambda b, i: (b, i, 0, 0)),
            pl.BlockSpec((H, C), lambda b, i: (0, 0)),
            pl.BlockSpec((TS * H, TS * C2 + C), lambda b, i: (0, 0)),
            pl.BlockSpec((H, 1), lambda b, i: (0, 0)),
            pl.BlockSpec((G * H, G * H), lambda b, i: (0, 0)),
            pl.BlockSpec((G * H, 1), lambda b, i: (0, 0)),
        ],
        out_specs=pl.BlockSpec((1, 1, H, TI), lambda b, i: (b, i, 0, 0)),
        out_shape=jax.ShapeDtypeStruct((B, V // TI, H, TI), jnp.float32),
        compiler_params=pltpu.CompilerParams(
            dimension_semantics=("parallel", "parallel")),
    )(xq, xq, adj, eT, WabT, W1blk, b1.reshape(H, 1), W2blk, b2G)


def _fin_body(TI, TS, zfq_ref, zq_ref, WaT_ref, WbT_ref, b3T_ref,
              W4sel_ref, b4_ref, out_ref):
    NG = zfq_ref.shape[1]
    V = NG * TI
    H2 = WaT_ref.shape[0]
    zfq = zfq_ref[0]
    PT = jnp.concatenate(
        [WaT_ref[...] @ zfq[g] for g in range(NG)], axis=1)
    QT = WbT_ref[...] @ zq_ref[0, 0] + b3T_ref[...]
    PTt = jnp.broadcast_to(PT[None], (TS, H2, V)).reshape(TS * H2, V)
    rows = []
    for sub in range(TI // TS):
        Qcol = jnp.concatenate(
            [QT[:, sub * TS + t:sub * TS + t + 1] for t in range(TS)],
            axis=0)
        Hb = jnp.maximum(PTt + Qcol, 0.0)
        ot = W4sel_ref[...] @ Hb + b4_ref[...]
        rows.append(jax.nn.sigmoid(ot))
    out_ref[0] = jnp.concatenate(rows, axis=0)


def _edge_predict(zq, lin3_W, lin3_b, out_W, out_b):
    B, NG, C, TIx = zq.shape
    V = NG * TIx
    H2 = lin3_W.shape[1]
    TI, TS = _TI, _TS
    WaT = lin3_W[:C].T
    WbT = lin3_W[C:].T
    W4sel = jnp.kron(jnp.eye(TS, dtype=jnp.float32), out_W.reshape(1, H2))
    return pl.pallas_call(
        functools.partial(_fin_body, TI, TS),
        grid=(B, V // TI),
        in_specs=[
            pl.BlockSpec((1, NG, C, TI), lambda b, i: (b, 0, 0, 0)),
            pl.BlockSpec((1, 1, C, TI), lambda b, i: (b, i, 0, 0)),
            pl.BlockSpec((H2, C), lambda b, i: (0, 0)),
            pl.BlockSpec((H2, C), lambda b, i: (0, 0)),
            pl.BlockSpec((H2, 1), lambda b, i: (0, 0)),
            pl.BlockSpec((TS, TS * H2), lambda b, i: (0, 0)),
            pl.BlockSpec((1, 1), lambda b, i: (0, 0)),
        ],
        out_specs=pl.BlockSpec((1, TI, V), lambda b, i: (b, i, 0)),
        out_shape=jax.ShapeDtypeStruct((B, V, V), jnp.float32),
        compiler_params=pltpu.CompilerParams(
            dimension_semantics=("parallel", "parallel")),
    )(zq, zq, WaT, WbT, lin3_b.reshape(H2, 1), W4sel, out_b.reshape(1, 1))


def kernel(adjacency, node_features, edge_attributes,
           ec1_W1, ec1_b1, ec1_W2, ec1_b2,
           ec2_W1, ec2_b1, ec2_W2, ec2_b2,
           lin3_W, lin3_b, out_W, out_b):
    B, V, C = node_features.shape
    TI = _TI
    xq = node_features.reshape(B, V // TI, TI, C).transpose(0, 1, 3, 2)
    eT = edge_attributes.transpose(0, 1, 3, 2)
    yq = _edge_conv(adjacency, xq, eT, ec1_W1, ec1_b1, ec1_W2, ec1_b2)
    zq = _edge_conv(adjacency, yq, eT, ec2_W1, ec2_b1, ec2_W2, ec2_b2)
    return _edge_predict(zq, lin3_W, lin3_b, out_W, out_b)
